# baseline (speedup 1.0000x reference)
import functools

import jax
import jax.numpy as jnp
from jax import lax
from jax.experimental import pallas as pl
from jax.experimental.pallas import tpu as pltpu
from jax.experimental.pallas import tpu_sc as plsc

VOCAB = 1000000
DIM = 64
BATCH = 16384
HIST = 50

NC = 2
NS = 16
NW = NC * NS

ROWS_PER_W = BATCH // NW
ROWS_PER_GATHER = 2
IDX_PER_GATHER = ROWS_PER_GATHER * HIST
CHUNKS = ROWS_PER_W // ROWS_PER_GATHER
NBUF = 4
UNROLL = 5
OUT_ST = 128

LANES = 128
NTILES = VOCAB // LANES
TAIL = VOCAB - NTILES * LANES
A_ITERS = (NTILES + NW - 1) // NW

_mesh = plsc.VectorSubcoreMesh(core_axis_name="c", subcore_axis_name="s")


@functools.partial(
    pl.kernel,
    mesh=_mesh,
    compiler_params=pltpu.CompilerParams(
        use_tc_tiling_on_sc=True, needs_layout_passes=False),
    out_type=jax.ShapeDtypeStruct((VOCAB, 2 * DIM), jnp.float32),
    scratch_types=[
        pltpu.VMEM((2, DIM, LANES), jnp.float32),
        pltpu.VMEM((2, LANES, 2 * DIM), jnp.float32),
        pltpu.VMEM((DIM, TAIL), jnp.float32),
        pltpu.SemaphoreType.DMA((2,)),
        pltpu.SemaphoreType.DMA((2,)),
    ],
)
def _repack_sc(tabt_hbm, tail_hbm, out_hbm, in_v, out_v, tail_v, in_sems,
               out_sems):
    wid = lax.axis_index("s") * NC + lax.axis_index("c")
    iota = lax.iota(jnp.int32, 16)

    def start_in(i, b):
        t = i * NW + wid
        @pl.when(t < NTILES)
        def _():
            col0 = pl.multiple_of(t * LANES, LANES)
            pltpu.async_copy(
                tabt_hbm.at[:, pl.ds(col0, LANES)], in_v.at[b],
                in_sems.at[b])

    start_in(0, 0)
    start_in(1, 1)

    def group(g, _):
        for b in range(2):
            i = g * 2 + b
            t = i * NW + wid

            @pl.when(t < NTILES)
            def _():
                pltpu.make_async_copy(
                    tabt_hbm.at[:, pl.ds(0, LANES)], in_v.at[b],
                    in_sems.at[b]).wait()
                @pl.when(i >= 2)
                def _():
                    pltpu.make_async_copy(
                        out_v.at[b],
                        out_hbm.at[pl.ds(0, LANES)],
                        out_sems.at[b]).wait()

                blk = in_v.at[b]
                ob = out_v.at[b]

                def j_body(j, _):
                    cols = jnp.full((16,), j, jnp.int32)
                    for k in range(4):
                        v = plsc.load_gather(blk, [iota + 16 * k, cols])
                        ob[j, pl.ds(16 * k, 16)] = v
                    return 0

                lax.fori_loop(0, LANES, j_body, 0)

                row0 = pl.multiple_of(t * LANES, LANES)
                pltpu.async_copy(
                    ob, out_hbm.at[pl.ds(row0, LANES)], out_sems.at[b])
                start_in(i + 2, b)
        return 0

    lax.fori_loop(0, (A_ITERS + 1) // 2, group, 0)

    for b in range(2):
        pltpu.make_async_copy(
            out_v.at[b], out_hbm.at[pl.ds(0, LANES)],
            out_sems.at[b]).wait()

    @pl.when(wid == 0)
    def _():
        pltpu.sync_copy(tail_hbm, tail_v)

        def j_body(j, _):
            cols = jnp.full((16,), j, jnp.int32)
            for k in range(4):
                v = plsc.load_gather(tail_v, [iota + 16 * k, cols])
                out_v[0, j, pl.ds(16 * k, 16)] = v
            return 0

        lax.fori_loop(0, TAIL, j_body, 0)
        pltpu.sync_copy(out_v.at[0, pl.ds(0, TAIL)],
                        out_hbm.at[pl.ds(NTILES * LANES, TAIL)])


@functools.partial(
    pl.kernel,
    mesh=_mesh,
    compiler_params=pltpu.CompilerParams(use_tc_tiling_on_sc=True),
    out_type=jax.ShapeDtypeStruct((BATCH, DIM), jnp.float32),
    scratch_types=[
        pltpu.VMEM((CHUNKS, IDX_PER_GATHER), jnp.int32),
        pltpu.VMEM((NBUF, IDX_PER_GATHER, 2 * DIM), jnp.float32),
        pltpu.VMEM((OUT_ST, DIM), jnp.float32),
        pltpu.SemaphoreType.DMA((NBUF,)),
    ],
)
def _cbow_sc(idx_hbm, table_hbm, out_hbm, idx_v, bufs_v, out_v, sems):
    wid = lax.axis_index("s") * NC + lax.axis_index("c")
    row0 = wid * ROWS_PER_W

    pltpu.sync_copy(idx_hbm.at[wid], idx_v)

    zero = jnp.zeros((16,), jnp.float32)

    for b in range(NBUF):
        pltpu.async_copy(table_hbm.at[idx_v.at[b]], bufs_v.at[b], sems.at[b])

    def group_body(g, _):
        for b in range(NBUF):
            c = g * NBUF + b
            buf = bufs_v.at[b]
            pltpu.make_async_copy(
                table_hbm.at[idx_v.at[c]], buf, sems.at[b]).wait()

            for r in range(ROWS_PER_GATHER):
                def h_body(h, accs, r=r, buf=buf):
                    a0, a1, a2, a3 = accs
                    for u in range(UNROLL):
                        hp = r * HIST + h * UNROLL + u
                        a0 = a0 + buf[hp, pl.ds(0, 16)]
                        a1 = a1 + buf[hp, pl.ds(16, 16)]
                        a2 = a2 + buf[hp, pl.ds(32, 16)]
                        a3 = a3 + buf[hp, pl.ds(48, 16)]
                    return (a0, a1, a2, a3)

                a0, a1, a2, a3 = lax.fori_loop(
                    0, HIST // UNROLL, h_body, (zero, zero, zero, zero))
                row = (c * ROWS_PER_GATHER + r) % OUT_ST
                out_v[row, pl.ds(0, 16)] = a0
                out_v[row, pl.ds(16, 16)] = a1
                out_v[row, pl.ds(32, 16)] = a2
                out_v[row, pl.ds(48, 16)] = a3

            nxt = c + NBUF
            @pl.when(nxt < CHUNKS)
            def _():
                pltpu.async_copy(
                    table_hbm.at[idx_v.at[nxt]], bufs_v.at[b], sems.at[b])

            done = (c + 1) * ROWS_PER_GATHER
            @pl.when(done % OUT_ST == 0)
            def _():
                off = pl.multiple_of(row0 + done - OUT_ST, OUT_ST)
                pltpu.sync_copy(out_v, out_hbm.at[pl.ds(off, OUT_ST)])
        return 0

    lax.fori_loop(0, CHUNKS // NBUF, group_body, 0)


def kernel(input_text, table):
    tabt = table.T
    tab128 = _repack_sc(tabt, tabt[:, NTILES * LANES:])
    idx3 = input_text.reshape(NW, CHUNKS, IDX_PER_GATHER)
    return _cbow_sc(idx3, tab128)


# speedup vs baseline: 1.2067x; 1.2067x over previous
import functools

import jax
import jax.numpy as jnp
from jax import lax
from jax.experimental import pallas as pl
from jax.experimental.pallas import tpu as pltpu
from jax.experimental.pallas import tpu_sc as plsc

VOCAB = 1000000
DIM = 64
BATCH = 16384
HIST = 50

NC = 2
NS = 16
NW = NC * NS

ROWS_PER_W = BATCH // NW
ROWS_PER_GATHER = 2
IDX_PER_GATHER = ROWS_PER_GATHER * HIST
CHUNKS = ROWS_PER_W // ROWS_PER_GATHER
NBUF = 4
UNROLL = 5
OUT_ST = 128

LANES = 128
NTILES = VOCAB // LANES
TAIL = VOCAB - NTILES * LANES
A_ITERS = (NTILES + NW - 1) // NW

_mesh = plsc.VectorSubcoreMesh(core_axis_name="c", subcore_axis_name="s")


@functools.partial(
    pl.kernel,
    mesh=_mesh,
    compiler_params=pltpu.CompilerParams(
        use_tc_tiling_on_sc=True, needs_layout_passes=False),
    out_type=jax.ShapeDtypeStruct((VOCAB, 2 * DIM), jnp.float32),
    scratch_types=[
        pltpu.VMEM((2, DIM, LANES), jnp.float32),
        pltpu.VMEM((2, LANES, 2 * DIM), jnp.float32),
        pltpu.VMEM((DIM, TAIL), jnp.float32),
        pltpu.SemaphoreType.DMA((2,)),
        pltpu.SemaphoreType.DMA((2,)),
    ],
)
def _repack_sc(tabt_hbm, tail_hbm, out_hbm, in_v, out_v, tail_v, in_sems,
               out_sems):
    wid = lax.axis_index("s") * NC + lax.axis_index("c")
    iota = lax.iota(jnp.int32, 16)

    def start_in(i, b):
        t = i * NW + wid
        @pl.when(t < NTILES)
        def _():
            col0 = pl.multiple_of(t * LANES, LANES)
            pltpu.async_copy(
                tabt_hbm.at[:, pl.ds(col0, LANES)], in_v.at[b],
                in_sems.at[b])

    start_in(0, 0)
    start_in(1, 1)

    def group(g, _):
        for b in range(2):
            i = g * 2 + b
            t = i * NW + wid

            @pl.when(t < NTILES)
            def _():
                pltpu.make_async_copy(
                    tabt_hbm.at[:, pl.ds(0, LANES)], in_v.at[b],
                    in_sems.at[b]).wait()
                @pl.when(i >= 2)
                def _():
                    pltpu.make_async_copy(
                        out_v.at[b],
                        out_hbm.at[pl.ds(0, LANES)],
                        out_sems.at[b]).wait()

                blk = in_v.at[b]
                ob = out_v.at[b]

                def j_body(j, _):
                    cols = jnp.full((16,), j, jnp.int32)
                    for k in range(4):
                        v = plsc.load_gather(blk, [iota + 16 * k, cols])
                        ob[j, pl.ds(16 * k, 16)] = v
                    return 0

                lax.fori_loop(0, LANES, j_body, 0)

                row0 = pl.multiple_of(t * LANES, LANES)
                pltpu.async_copy(
                    ob, out_hbm.at[pl.ds(row0, LANES)], out_sems.at[b])
                start_in(i + 2, b)
        return 0

    lax.fori_loop(0, (A_ITERS + 1) // 2, group, 0)

    for b in range(2):
        pltpu.make_async_copy(
            out_v.at[b], out_hbm.at[pl.ds(0, LANES)],
            out_sems.at[b]).wait()

    @pl.when(wid == 0)
    def _():
        pltpu.sync_copy(tail_hbm, tail_v)

        def j_body(j, _):
            cols = jnp.full((16,), j, jnp.int32)
            for k in range(4):
                v = plsc.load_gather(tail_v, [iota + 16 * k, cols])
                out_v[0, j, pl.ds(16 * k, 16)] = v
            return 0

        lax.fori_loop(0, TAIL, j_body, 0)
        pltpu.sync_copy(out_v.at[0, pl.ds(0, TAIL)],
                        out_hbm.at[pl.ds(NTILES * LANES, TAIL)])


@functools.partial(
    pl.kernel,
    mesh=_mesh,
    compiler_params=pltpu.CompilerParams(use_tc_tiling_on_sc=True),
    out_type=jax.ShapeDtypeStruct((BATCH, DIM), jnp.float32),
    scratch_types=[
        pltpu.VMEM((CHUNKS, IDX_PER_GATHER), jnp.int32),
        pltpu.VMEM((NBUF, IDX_PER_GATHER, 2 * DIM), jnp.float32),
        pltpu.VMEM((OUT_ST, DIM), jnp.float32),
        pltpu.SemaphoreType.DMA((NBUF,)),
    ],
)
def _cbow_sc(idx_hbm, table_hbm, out_hbm, idx_v, bufs_v, out_v, sems):
    wid = lax.axis_index("s") * NC + lax.axis_index("c")
    row0 = wid * ROWS_PER_W

    pltpu.sync_copy(idx_hbm.at[wid], idx_v)

    zero = jnp.zeros((16,), jnp.float32)

    for b in range(NBUF):
        pltpu.async_copy(table_hbm.at[idx_v.at[b]], bufs_v.at[b], sems.at[b])

    def group_body(g, _):
        for b in range(NBUF):
            c = g * NBUF + b
            buf = bufs_v.at[b]
            pltpu.make_async_copy(
                table_hbm.at[idx_v.at[c]], buf, sems.at[b]).wait()

            for r in range(ROWS_PER_GATHER):
                def h_body(h, accs, r=r, buf=buf):
                    a0, a1, a2, a3 = accs
                    for u in range(UNROLL):
                        hp = r * HIST + h * UNROLL + u
                        a0 = a0 + buf[hp, pl.ds(0, 16)]
                        a1 = a1 + buf[hp, pl.ds(16, 16)]
                        a2 = a2 + buf[hp, pl.ds(32, 16)]
                        a3 = a3 + buf[hp, pl.ds(48, 16)]
                    return (a0, a1, a2, a3)

                a0, a1, a2, a3 = lax.fori_loop(
                    0, HIST // UNROLL, h_body, (zero, zero, zero, zero))
                row = (c * ROWS_PER_GATHER + r) % OUT_ST
                out_v[row, pl.ds(0, 16)] = a0
                out_v[row, pl.ds(16, 16)] = a1
                out_v[row, pl.ds(32, 16)] = a2
                out_v[row, pl.ds(48, 16)] = a3

            nxt = c + NBUF
            @pl.when(nxt < CHUNKS)
            def _():
                pltpu.async_copy(
                    table_hbm.at[idx_v.at[nxt]], bufs_v.at[b], sems.at[b])

            done = (c + 1) * ROWS_PER_GATHER
            @pl.when(done % OUT_ST == 0)
            def _():
                off = pl.multiple_of(row0 + done - OUT_ST, OUT_ST)
                pltpu.sync_copy(out_v, out_hbm.at[pl.ds(off, OUT_ST)])
        return 0

    lax.fori_loop(0, CHUNKS // NBUF, group_body, 0)


TBLK = 512


def _tc_repack_body(x_ref, o_ref):
    o_ref[:, pl.ds(0, DIM)] = x_ref[...].T


_tc_repack = pl.pallas_call(
    _tc_repack_body,
    grid=((VOCAB + TBLK - 1) // TBLK,),
    in_specs=[pl.BlockSpec((DIM, TBLK), lambda i: (0, i))],
    out_specs=pl.BlockSpec((TBLK, 2 * DIM), lambda i: (i, 0)),
    out_shape=jax.ShapeDtypeStruct((VOCAB, 2 * DIM), jnp.float32),
)


def kernel(input_text, table):
    tab128 = _tc_repack(table.T)
    idx3 = input_text.reshape(NW, CHUNKS, IDX_PER_GATHER)
    return _cbow_sc(idx3, tab128)


# speedup vs baseline: 2.4631x; 2.0412x over previous
import functools

import jax
import jax.numpy as jnp
from jax import lax
from jax.experimental import pallas as pl
from jax.experimental.pallas import tpu as pltpu
from jax.experimental.pallas import tpu_sc as plsc

VOCAB = 1000000
DIM = 64
BATCH = 16384
HIST = 50

NC = 2
NS = 16
NW = NC * NS

ROWS_PER_W = BATCH // NW
ROWS_PER_GATHER = 2
IDX_PER_GATHER = ROWS_PER_GATHER * HIST
CHUNKS = ROWS_PER_W // ROWS_PER_GATHER
NBUF = 4
UNROLL = 5
OUT_ST = 128

LANES = 128
NTILES = VOCAB // LANES
TAIL = VOCAB - NTILES * LANES
A_ITERS = (NTILES + NW - 1) // NW

_mesh = plsc.VectorSubcoreMesh(core_axis_name="c", subcore_axis_name="s")


@functools.partial(
    pl.kernel,
    mesh=_mesh,
    compiler_params=pltpu.CompilerParams(
        use_tc_tiling_on_sc=True, needs_layout_passes=False),
    out_type=jax.ShapeDtypeStruct((VOCAB, 2 * DIM), jnp.float32),
    scratch_types=[
        pltpu.VMEM((2, DIM, LANES), jnp.float32),
        pltpu.VMEM((2, LANES, 2 * DIM), jnp.float32),
        pltpu.VMEM((DIM, TAIL), jnp.float32),
        pltpu.SemaphoreType.DMA((2,)),
        pltpu.SemaphoreType.DMA((2,)),
    ],
)
def _repack_sc(tabt_hbm, tail_hbm, out_hbm, in_v, out_v, tail_v, in_sems,
               out_sems):
    wid = lax.axis_index("s") * NC + lax.axis_index("c")
    iota = lax.iota(jnp.int32, 16)

    def start_in(i, b):
        t = i * NW + wid
        @pl.when(t < NTILES)
        def _():
            col0 = pl.multiple_of(t * LANES, LANES)
            pltpu.async_copy(
                tabt_hbm.at[:, pl.ds(col0, LANES)], in_v.at[b],
                in_sems.at[b])

    start_in(0, 0)
    start_in(1, 1)

    def group(g, _):
        for b in range(2):
            i = g * 2 + b
            t = i * NW + wid

            @pl.when(t < NTILES)
            def _():
                pltpu.make_async_copy(
                    tabt_hbm.at[:, pl.ds(0, LANES)], in_v.at[b],
                    in_sems.at[b]).wait()
                @pl.when(i >= 2)
                def _():
                    pltpu.make_async_copy(
                        out_v.at[b],
                        out_hbm.at[pl.ds(0, LANES)],
                        out_sems.at[b]).wait()

                blk = in_v.at[b]
                ob = out_v.at[b]

                def j_body(j, _):
                    cols = jnp.full((16,), j, jnp.int32)
                    for k in range(4):
                        v = plsc.load_gather(blk, [iota + 16 * k, cols])
                        ob[j, pl.ds(16 * k, 16)] = v
                    return 0

                lax.fori_loop(0, LANES, j_body, 0)

                row0 = pl.multiple_of(t * LANES, LANES)
                pltpu.async_copy(
                    ob, out_hbm.at[pl.ds(row0, LANES)], out_sems.at[b])
                start_in(i + 2, b)
        return 0

    lax.fori_loop(0, (A_ITERS + 1) // 2, group, 0)

    for b in range(2):
        pltpu.make_async_copy(
            out_v.at[b], out_hbm.at[pl.ds(0, LANES)],
            out_sems.at[b]).wait()

    @pl.when(wid == 0)
    def _():
        pltpu.sync_copy(tail_hbm, tail_v)

        def j_body(j, _):
            cols = jnp.full((16,), j, jnp.int32)
            for k in range(4):
                v = plsc.load_gather(tail_v, [iota + 16 * k, cols])
                out_v[0, j, pl.ds(16 * k, 16)] = v
            return 0

        lax.fori_loop(0, TAIL, j_body, 0)
        pltpu.sync_copy(out_v.at[0, pl.ds(0, TAIL)],
                        out_hbm.at[pl.ds(NTILES * LANES, TAIL)])


@functools.partial(
    pl.kernel,
    mesh=_mesh,
    compiler_params=pltpu.CompilerParams(use_tc_tiling_on_sc=True),
    out_type=jax.ShapeDtypeStruct((BATCH, DIM), jnp.float32),
    scratch_types=[
        pltpu.VMEM((CHUNKS, IDX_PER_GATHER), jnp.int32),
        pltpu.VMEM((NBUF, IDX_PER_GATHER, 2 * DIM), jnp.float32),
        pltpu.VMEM((OUT_ST, DIM), jnp.float32),
        pltpu.SemaphoreType.DMA((NBUF,)),
    ],
)
def _cbow_sc(idx_hbm, table_hbm, out_hbm, idx_v, bufs_v, out_v, sems):
    wid = lax.axis_index("s") * NC + lax.axis_index("c")
    row0 = wid * ROWS_PER_W

    pltpu.sync_copy(idx_hbm.at[wid], idx_v)

    zero = jnp.zeros((16,), jnp.float32)

    for b in range(NBUF):
        pltpu.async_copy(table_hbm.at[idx_v.at[b]], bufs_v.at[b], sems.at[b])

    def group_body(g, _):
        for b in range(NBUF):
            c = g * NBUF + b
            buf = bufs_v.at[b]
            pltpu.make_async_copy(
                table_hbm.at[idx_v.at[c]], buf, sems.at[b]).wait()

            for r in range(ROWS_PER_GATHER):
                def h_body(h, accs, r=r, buf=buf):
                    a0, a1, a2, a3 = accs
                    for u in range(UNROLL):
                        hp = r * HIST + h * UNROLL + u
                        a0 = a0 + buf[hp, pl.ds(0, 16)]
                        a1 = a1 + buf[hp, pl.ds(16, 16)]
                        a2 = a2 + buf[hp, pl.ds(32, 16)]
                        a3 = a3 + buf[hp, pl.ds(48, 16)]
                    return (a0, a1, a2, a3)

                a0, a1, a2, a3 = lax.fori_loop(
                    0, HIST // UNROLL, h_body, (zero, zero, zero, zero))
                row = (c * ROWS_PER_GATHER + r) % OUT_ST
                out_v[row, pl.ds(0, 16)] = a0
                out_v[row, pl.ds(16, 16)] = a1
                out_v[row, pl.ds(32, 16)] = a2
                out_v[row, pl.ds(48, 16)] = a3

            nxt = c + NBUF
            @pl.when(nxt < CHUNKS)
            def _():
                pltpu.async_copy(
                    table_hbm.at[idx_v.at[nxt]], bufs_v.at[b], sems.at[b])

            done = (c + 1) * ROWS_PER_GATHER
            @pl.when(done % OUT_ST == 0)
            def _():
                off = pl.multiple_of(row0 + done - OUT_ST, OUT_ST)
                pltpu.sync_copy(out_v, out_hbm.at[pl.ds(off, OUT_ST)])
        return 0

    lax.fori_loop(0, CHUNKS // NBUF, group_body, 0)


TBLK = 2048


def _tc_repack_body(x_ref, eye_ref, o_ref):
    o_ref[:, pl.ds(0, DIM)] = lax.dot_general(
        x_ref[...], eye_ref[...],
        dimension_numbers=(((0,), (0,)), ((), ())),
        preferred_element_type=jnp.float32)


_tc_repack = pl.pallas_call(
    _tc_repack_body,
    grid=((VOCAB + TBLK - 1) // TBLK,),
    in_specs=[pl.BlockSpec((DIM, TBLK), lambda i: (0, i)),
              pl.BlockSpec((DIM, DIM), lambda i: (0, 0))],
    out_specs=pl.BlockSpec((TBLK, 2 * DIM), lambda i: (i, 0)),
    out_shape=jax.ShapeDtypeStruct((VOCAB, 2 * DIM), jnp.float32),
)


def kernel(input_text, table):
    tab128 = _tc_repack(table.T, jnp.eye(DIM, dtype=jnp.float32))
    idx3 = input_text.reshape(NW, CHUNKS, IDX_PER_GATHER)
    return _cbow_sc(idx3, tab128)


# speedup vs baseline: 3.8228x; 1.5520x over previous
import functools

import jax
import jax.numpy as jnp
from jax import lax
from jax.experimental import pallas as pl
from jax.experimental.pallas import tpu as pltpu
from jax.experimental.pallas import tpu_sc as plsc

VOCAB = 1000000
DIM = 64
BATCH = 16384
HIST = 50

NC = 2
NS = 16
NW = NC * NS

ROWS_PER_W = BATCH // NW
ROWS_PER_GATHER = 2
IDX_PER_GATHER = ROWS_PER_GATHER * HIST
CHUNKS = ROWS_PER_W // ROWS_PER_GATHER
NBUF = 4
UNROLL = 5
OUT_ST = 128

LANES = 128
NTILES = VOCAB // LANES
TAIL = VOCAB - NTILES * LANES
A_ITERS = (NTILES + NW - 1) // NW

_mesh = plsc.VectorSubcoreMesh(core_axis_name="c", subcore_axis_name="s")


@functools.partial(
    pl.kernel,
    mesh=_mesh,
    compiler_params=pltpu.CompilerParams(
        use_tc_tiling_on_sc=True, needs_layout_passes=False),
    out_type=jax.ShapeDtypeStruct((VOCAB, 2 * DIM), jnp.float32),
    scratch_types=[
        pltpu.VMEM((2, DIM, LANES), jnp.float32),
        pltpu.VMEM((2, LANES, 2 * DIM), jnp.float32),
        pltpu.VMEM((DIM, TAIL), jnp.float32),
        pltpu.SemaphoreType.DMA((2,)),
        pltpu.SemaphoreType.DMA((2,)),
    ],
)
def _repack_sc(tabt_hbm, tail_hbm, out_hbm, in_v, out_v, tail_v, in_sems,
               out_sems):
    wid = lax.axis_index("s") * NC + lax.axis_index("c")
    iota = lax.iota(jnp.int32, 16)

    def start_in(i, b):
        t = i * NW + wid
        @pl.when(t < NTILES)
        def _():
            col0 = pl.multiple_of(t * LANES, LANES)
            pltpu.async_copy(
                tabt_hbm.at[:, pl.ds(col0, LANES)], in_v.at[b],
                in_sems.at[b])

    start_in(0, 0)
    start_in(1, 1)

    def group(g, _):
        for b in range(2):
            i = g * 2 + b
            t = i * NW + wid

            @pl.when(t < NTILES)
            def _():
                pltpu.make_async_copy(
                    tabt_hbm.at[:, pl.ds(0, LANES)], in_v.at[b],
                    in_sems.at[b]).wait()
                @pl.when(i >= 2)
                def _():
                    pltpu.make_async_copy(
                        out_v.at[b],
                        out_hbm.at[pl.ds(0, LANES)],
                        out_sems.at[b]).wait()

                blk = in_v.at[b]
                ob = out_v.at[b]

                def j_body(j, _):
                    cols = jnp.full((16,), j, jnp.int32)
                    for k in range(4):
                        v = plsc.load_gather(blk, [iota + 16 * k, cols])
                        ob[j, pl.ds(16 * k, 16)] = v
                    return 0

                lax.fori_loop(0, LANES, j_body, 0)

                row0 = pl.multiple_of(t * LANES, LANES)
                pltpu.async_copy(
                    ob, out_hbm.at[pl.ds(row0, LANES)], out_sems.at[b])
                start_in(i + 2, b)
        return 0

    lax.fori_loop(0, (A_ITERS + 1) // 2, group, 0)

    for b in range(2):
        pltpu.make_async_copy(
            out_v.at[b], out_hbm.at[pl.ds(0, LANES)],
            out_sems.at[b]).wait()

    @pl.when(wid == 0)
    def _():
        pltpu.sync_copy(tail_hbm, tail_v)

        def j_body(j, _):
            cols = jnp.full((16,), j, jnp.int32)
            for k in range(4):
                v = plsc.load_gather(tail_v, [iota + 16 * k, cols])
                out_v[0, j, pl.ds(16 * k, 16)] = v
            return 0

        lax.fori_loop(0, TAIL, j_body, 0)
        pltpu.sync_copy(out_v.at[0, pl.ds(0, TAIL)],
                        out_hbm.at[pl.ds(NTILES * LANES, TAIL)])


@functools.partial(
    pl.kernel,
    mesh=_mesh,
    compiler_params=pltpu.CompilerParams(use_tc_tiling_on_sc=True),
    out_type=jax.ShapeDtypeStruct((BATCH, DIM), jnp.float32),
    scratch_types=[
        pltpu.VMEM((CHUNKS, IDX_PER_GATHER), jnp.int32),
        pltpu.VMEM((NBUF, IDX_PER_GATHER, 2 * DIM), jnp.float32),
        pltpu.VMEM((OUT_ST, DIM), jnp.float32),
        pltpu.SemaphoreType.DMA((NBUF,)),
    ],
)
def _cbow_sc(idx_hbm, table_hbm, out_hbm, idx_v, bufs_v, out_v, sems):
    wid = lax.axis_index("s") * NC + lax.axis_index("c")
    row0 = wid * ROWS_PER_W

    pltpu.sync_copy(idx_hbm.at[wid], idx_v)

    zero = jnp.zeros((16,), jnp.float32)

    for b in range(NBUF):
        pltpu.async_copy(table_hbm.at[idx_v.at[b]], bufs_v.at[b], sems.at[b])

    def group_body(g, _):
        for b in range(NBUF):
            c = g * NBUF + b
            buf = bufs_v.at[b]
            pltpu.make_async_copy(
                table_hbm.at[idx_v.at[c]], buf, sems.at[b]).wait()

            for r in range(ROWS_PER_GATHER):
                def h_body(h, accs, r=r, buf=buf):
                    a0, a1, a2, a3 = accs
                    for u in range(UNROLL):
                        hp = r * HIST + h * UNROLL + u
                        a0 = a0 + buf[hp, pl.ds(0, 16)]
                        a1 = a1 + buf[hp, pl.ds(16, 16)]
                        a2 = a2 + buf[hp, pl.ds(32, 16)]
                        a3 = a3 + buf[hp, pl.ds(48, 16)]
                    return (a0, a1, a2, a3)

                a0, a1, a2, a3 = lax.fori_loop(
                    0, HIST // UNROLL, h_body, (zero, zero, zero, zero))
                row = (c * ROWS_PER_GATHER + r) % OUT_ST
                out_v[row, pl.ds(0, 16)] = a0
                out_v[row, pl.ds(16, 16)] = a1
                out_v[row, pl.ds(32, 16)] = a2
                out_v[row, pl.ds(48, 16)] = a3

            nxt = c + NBUF
            @pl.when(nxt < CHUNKS)
            def _():
                pltpu.async_copy(
                    table_hbm.at[idx_v.at[nxt]], bufs_v.at[b], sems.at[b])

            done = (c + 1) * ROWS_PER_GATHER
            @pl.when(done % OUT_ST == 0)
            def _():
                off = pl.multiple_of(row0 + done - OUT_ST, OUT_ST)
                pltpu.sync_copy(out_v, out_hbm.at[pl.ds(off, OUT_ST)])
        return 0

    lax.fori_loop(0, CHUNKS // NBUF, group_body, 0)


TBLK = 32768


def _tc_repack_body(x_ref, eye_ref, o_ref):
    o_ref[:, pl.ds(0, DIM)] = lax.dot_general(
        x_ref[...], eye_ref[...],
        dimension_numbers=(((0,), (0,)), ((), ())),
        preferred_element_type=jnp.float32)


_tc_repack = pl.pallas_call(
    _tc_repack_body,
    grid=((VOCAB + TBLK - 1) // TBLK,),
    in_specs=[pl.BlockSpec((DIM, TBLK), lambda i: (0, i)),
              pl.BlockSpec((DIM, DIM), lambda i: (0, 0))],
    out_specs=pl.BlockSpec((TBLK, 2 * DIM), lambda i: (i, 0)),
    out_shape=jax.ShapeDtypeStruct((VOCAB, 2 * DIM), jnp.float32),
)


def kernel(input_text, table):
    tab128 = _tc_repack(table.T, jnp.eye(DIM, dtype=jnp.float32))
    idx3 = input_text.reshape(NW, CHUNKS, IDX_PER_GATHER)
    return _cbow_sc(idx3, tab128)
